# baseline (speedup 1.0000x reference)
import jax
import jax.numpy as jnp
import numpy as np
from jax import lax
from jax.experimental import pallas as pl
from jax.experimental.pallas import tpu as pltpu
from jax.experimental.pallas import tpu_sc as plsc

_NUM_NODES = 100000
_NUM_EDGES = 3200000
_NUM_POSSIBLE = _NUM_NODES**2 - _NUM_NODES
_NUM_NONEDGES = _NUM_POSSIBLE - _NUM_EDGES
_EPS = float(-np.log(1.0 - _NUM_EDGES / _NUM_POSSIBLE))
_NEG_SCALE = float(_NUM_NONEDGES) / float(_NUM_EDGES)

_NC, _NS, _L = 2, 16, 16
_NW = _NC * _NS
_B = 262144
_TOT = 2 * _B
_PER_W = _TOT // _NW
_CHUNK = 128
_NCHUNK = _PER_W // _CHUNK
_D = 128
_KD = _D // _L


def _sc_dots_body(emb_hbm, li_hbm, ri_hbm, dots_hbm,
                  idx_l, idx_r, rows_l0, rows_r0, rows_l1, rows_r1,
                  rows_l2, rows_r2, rows_l3, rows_r3,
                  part, dots_v, sem0, sem1, sem2, sem3):
    wid = lax.axis_index("s") * _NC + lax.axis_index("c")
    base = wid * _PER_W
    pltpu.sync_copy(li_hbm.at[pl.ds(base, _PER_W)], idx_l)
    pltpu.sync_copy(ri_hbm.at[pl.ds(base, _PER_W)], idx_r)

    def _issue(c, rl, rr, sem):
        pltpu.async_copy(emb_hbm.at[idx_l.at[pl.ds(c * _CHUNK, _CHUNK)]], rl, sem)
        pltpu.async_copy(emb_hbm.at[idx_r.at[pl.ds(c * _CHUNK, _CHUNK)]], rr, sem)

    def _wait(c, rl, rr, sem):
        pltpu.make_async_copy(
            emb_hbm.at[idx_l.at[pl.ds(c * _CHUNK, _CHUNK)]], rl, sem).wait()
        pltpu.make_async_copy(
            emb_hbm.at[idx_r.at[pl.ds(c * _CHUNK, _CHUNK)]], rr, sem).wait()

    def _compute(c, rl, rr):
        @pl.loop(0, _CHUNK, unroll=8)
        def _pair(p):
            s0 = pl.ds(0, _L)
            s1 = pl.ds(_L, _L)
            acc0 = plsc.bitcast(rl[p, s0], jnp.bfloat16) * plsc.bitcast(rr[p, s0], jnp.bfloat16)
            acc1 = plsc.bitcast(rl[p, s1], jnp.bfloat16) * plsc.bitcast(rr[p, s1], jnp.bfloat16)
            for k in range(2, _D // (2 * _L)):
                s = pl.ds(k * _L, _L)
                m = plsc.bitcast(rl[p, s], jnp.bfloat16) * plsc.bitcast(rr[p, s], jnp.bfloat16)
                if k % 2 == 0:
                    acc0 = acc0 + m
                else:
                    acc1 = acc1 + m
            a, b = plsc.unpack(acc0 + acc1, format=plsc.PackFormat.INTERLEAVED)
            part[pl.ds(p * _L, _L)] = a + b

        @pl.loop(0, _CHUNK // _L, unroll=2)
        def _grp(g):
            flat = (g * _L + lax.iota(jnp.int32, _L)) * _L
            s = plsc.load_gather(part, [flat])
            for j in range(1, _L):
                s = s + plsc.load_gather(part, [flat + j])
            dots_v[pl.ds(g * _L, _L)] = s

        pltpu.sync_copy(dots_v, dots_hbm.at[pl.ds(base + c * _CHUNK, _CHUNK)])

    bufs = ((rows_l0, rows_r0, sem0), (rows_l1, rows_r1, sem1))
    del rows_l2, rows_r2, rows_l3, rows_r3, sem2, sem3
    nbuf = len(bufs)
    for j in range(nbuf - 1):
        _issue(j, *bufs[j])

    @pl.loop(0, _NCHUNK, step=nbuf)
    def _c(c):
        _issue(c + nbuf - 1, *bufs[nbuf - 1])
        for j, (rl, rr, sem) in enumerate(bufs):
            cc = c + j
            _wait(cc, rl, rr, sem)
            _compute(cc, rl, rr)
            if j < nbuf - 1:
                nxt = cc + nbuf

                @pl.when(nxt < _NCHUNK)
                def _refill(nxt=nxt, rl=rl, rr=rr, sem=sem):
                    _issue(nxt, rl, rr, sem)


def _sc_dots(emb, li, ri):
    mesh = plsc.VectorSubcoreMesh(core_axis_name="c", subcore_axis_name="s",
                                  num_cores=_NC, num_subcores=_NS)
    return pl.kernel(
        _sc_dots_body,
        out_type=jax.ShapeDtypeStruct((_TOT,), jnp.float32),
        mesh=mesh,
        scratch_types=[
            pltpu.VMEM((_PER_W,), jnp.int32),
            pltpu.VMEM((_PER_W,), jnp.int32),
            pltpu.VMEM((_CHUNK, _D // 2), jnp.int32),
            pltpu.VMEM((_CHUNK, _D // 2), jnp.int32),
            pltpu.VMEM((_CHUNK, _D // 2), jnp.int32),
            pltpu.VMEM((_CHUNK, _D // 2), jnp.int32),
            pltpu.VMEM((_CHUNK, _D // 2), jnp.int32),
            pltpu.VMEM((_CHUNK, _D // 2), jnp.int32),
            pltpu.VMEM((_CHUNK, _D // 2), jnp.int32),
            pltpu.VMEM((_CHUNK, _D // 2), jnp.int32),
            pltpu.VMEM((_CHUNK * _L,), jnp.float32),
            pltpu.VMEM((_CHUNK,), jnp.float32),
            pltpu.SemaphoreType.DMA,
            pltpu.SemaphoreType.DMA,
            pltpu.SemaphoreType.DMA,
            pltpu.SemaphoreType.DMA,
        ],
        compiler_params=pltpu.CompilerParams(needs_layout_passes=False,
                                             use_tc_tiling_on_sc=False),
    )(emb, li, ri)


_PACK_ROWS = 10000


def _pack_tc_body(x_ref, o_ref):
    bits = lax.bitcast_convert_type(x_ref[...], jnp.uint32)
    one = jnp.uint32(1)
    half = jnp.uint32(0x7FFF)
    lo = (bits[:, :_D // 2] + half + ((bits[:, :_D // 2] >> 16) & one)) >> 16
    hi = (bits[:, _D // 2:] + half + ((bits[:, _D // 2:] >> 16) & one)) >> 16
    o_ref[...] = lax.bitcast_convert_type(lo | (hi << 16), jnp.int32)


def _pack_tc(emb):
    return pl.pallas_call(
        _pack_tc_body,
        out_shape=jax.ShapeDtypeStruct((_NUM_NODES, _D // 2), jnp.int32),
        grid=(_NUM_NODES // _PACK_ROWS,),
        in_specs=[pl.BlockSpec((_PACK_ROWS, _D), lambda i: (i, 0))],
        out_specs=pl.BlockSpec((_PACK_ROWS, _D // 2), lambda i: (i, 0)),
    )(emb)


def _loss_tc_body(e_ref, z_ref, o_ref):
    e = e_ref[...]
    z = z_ref[...]
    loss_edges = -jnp.mean(jnp.log1p(-jnp.exp(-_EPS - e)))
    loss_non = jnp.mean(z)
    o_ref[0, 0] = (loss_edges + _NEG_SCALE * loss_non) / (1.0 + _NEG_SCALE)


def _loss_tc(edge_dots, non_dots):
    return pl.pallas_call(
        _loss_tc_body,
        out_shape=jax.ShapeDtypeStruct((1, 1), jnp.float32),
        out_specs=pl.BlockSpec(memory_space=pltpu.SMEM),
    )(edge_dots, non_dots)


@jax.jit
def kernel(emb, ones_idx, zeros_idx):
    li = jnp.concatenate([ones_idx[:, 0], zeros_idx[:, 0]]).astype(jnp.int32)
    ri = jnp.concatenate([ones_idx[:, 1], zeros_idx[:, 1]]).astype(jnp.int32)
    dots = _sc_dots(_pack_tc(emb), li, ri)
    edge_dots = dots[:_B].reshape(_B // _D, _D)
    non_dots = dots[_B:].reshape(_B // _D, _D)
    return _loss_tc(edge_dots, non_dots)[0, 0]


# speedup vs baseline: 1.1265x; 1.1265x over previous
import jax
import jax.numpy as jnp
import numpy as np
from jax import lax
from jax.experimental import pallas as pl
from jax.experimental.pallas import tpu as pltpu
from jax.experimental.pallas import tpu_sc as plsc

_NUM_NODES = 100000
_NUM_EDGES = 3200000
_NUM_POSSIBLE = _NUM_NODES**2 - _NUM_NODES
_NUM_NONEDGES = _NUM_POSSIBLE - _NUM_EDGES
_EPS = float(-np.log(1.0 - _NUM_EDGES / _NUM_POSSIBLE))
_NEG_SCALE = float(_NUM_NONEDGES) / float(_NUM_EDGES)

_NC, _NS, _L = 2, 16, 16
_NW = _NC * _NS
_B = 262144
_TOT = 2 * _B
_PER_W = _TOT // _NW
_CHUNK = 128
_NCHUNK = _PER_W // _CHUNK
_D = 128
_KD = _D // _L


def _sc_dots_body(emb_hbm, li_hbm, ri_hbm, dots_hbm,
                  idx_l, idx_r, rows_l0, rows_r0, rows_l1, rows_r1,
                  rows_l2, rows_r2, rows_l3, rows_r3,
                  part, dots_v, sem0, sem1, sem2, sem3):
    wid = lax.axis_index("s") * _NC + lax.axis_index("c")
    base = wid * _PER_W
    pltpu.sync_copy(li_hbm.at[pl.ds(base, _PER_W)], idx_l)
    pltpu.sync_copy(ri_hbm.at[pl.ds(base, _PER_W)], idx_r)

    def _issue(c, rl, rr, sem):
        pltpu.async_copy(emb_hbm.at[idx_l.at[pl.ds(c * _CHUNK, _CHUNK)]], rl, sem)
        pltpu.async_copy(emb_hbm.at[idx_r.at[pl.ds(c * _CHUNK, _CHUNK)]], rr, sem)

    def _wait(c, rl, rr, sem):
        pltpu.make_async_copy(
            emb_hbm.at[idx_l.at[pl.ds(c * _CHUNK, _CHUNK)]], rl, sem).wait()
        pltpu.make_async_copy(
            emb_hbm.at[idx_r.at[pl.ds(c * _CHUNK, _CHUNK)]], rr, sem).wait()

    def _compute(c, rl, rr):
        @pl.loop(0, _CHUNK, unroll=8)
        def _pair(p):
            s0 = pl.ds(0, _L)
            s1 = pl.ds(_L, _L)
            acc0 = plsc.bitcast(rl[p, s0], jnp.bfloat16) * plsc.bitcast(rr[p, s0], jnp.bfloat16)
            acc1 = plsc.bitcast(rl[p, s1], jnp.bfloat16) * plsc.bitcast(rr[p, s1], jnp.bfloat16)
            for k in range(2, _D // (2 * _L)):
                s = pl.ds(k * _L, _L)
                m = plsc.bitcast(rl[p, s], jnp.bfloat16) * plsc.bitcast(rr[p, s], jnp.bfloat16)
                if k % 2 == 0:
                    acc0 = acc0 + m
                else:
                    acc1 = acc1 + m
            a, b = plsc.unpack(acc0 + acc1, format=plsc.PackFormat.INTERLEAVED)
            part[pl.ds(p * _L, _L)] = a + b

        @pl.loop(0, _CHUNK // _L, unroll=2)
        def _grp(g):
            flat = (g * _L + lax.iota(jnp.int32, _L)) * _L
            s = plsc.load_gather(part, [flat])
            for j in range(1, _L):
                s = s + plsc.load_gather(part, [flat + j])
            dots_v[pl.ds(g * _L, _L)] = s

        pltpu.sync_copy(dots_v, dots_hbm.at[pl.ds(base + c * _CHUNK, _CHUNK)])

    bufs = ((rows_l0, rows_r0, sem0), (rows_l1, rows_r1, sem1))
    del rows_l2, rows_r2, rows_l3, rows_r3, sem2, sem3
    nbuf = len(bufs)
    for j in range(nbuf - 1):
        _issue(j, *bufs[j])

    @pl.loop(0, _NCHUNK, step=nbuf)
    def _c(c):
        _issue(c + nbuf - 1, *bufs[nbuf - 1])
        for j, (rl, rr, sem) in enumerate(bufs):
            cc = c + j
            _wait(cc, rl, rr, sem)
            _compute(cc, rl, rr)
            if j < nbuf - 1:
                nxt = cc + nbuf

                @pl.when(nxt < _NCHUNK)
                def _refill(nxt=nxt, rl=rl, rr=rr, sem=sem):
                    _issue(nxt, rl, rr, sem)


def _sc_dots(emb, li, ri):
    mesh = plsc.VectorSubcoreMesh(core_axis_name="c", subcore_axis_name="s",
                                  num_cores=_NC, num_subcores=_NS)
    return pl.kernel(
        _sc_dots_body,
        out_type=jax.ShapeDtypeStruct((_TOT,), jnp.float32),
        mesh=mesh,
        scratch_types=[
            pltpu.VMEM((_PER_W,), jnp.int32),
            pltpu.VMEM((_PER_W,), jnp.int32),
            pltpu.VMEM((_CHUNK, _D // 2), jnp.int32),
            pltpu.VMEM((_CHUNK, _D // 2), jnp.int32),
            pltpu.VMEM((_CHUNK, _D // 2), jnp.int32),
            pltpu.VMEM((_CHUNK, _D // 2), jnp.int32),
            pltpu.VMEM((_CHUNK, _D // 2), jnp.int32),
            pltpu.VMEM((_CHUNK, _D // 2), jnp.int32),
            pltpu.VMEM((_CHUNK, _D // 2), jnp.int32),
            pltpu.VMEM((_CHUNK, _D // 2), jnp.int32),
            pltpu.VMEM((_CHUNK * _L,), jnp.float32),
            pltpu.VMEM((_CHUNK,), jnp.float32),
            pltpu.SemaphoreType.DMA,
            pltpu.SemaphoreType.DMA,
            pltpu.SemaphoreType.DMA,
            pltpu.SemaphoreType.DMA,
        ],
        compiler_params=pltpu.CompilerParams(needs_layout_passes=False,
                                             use_tc_tiling_on_sc=False),
    )(emb, li, ri)


_PACK_ROWS = 10000


def _pack_tc_body(x_ref, o_ref):
    bits = lax.bitcast_convert_type(x_ref[...], jnp.uint32)
    one = jnp.uint32(1)
    half = jnp.uint32(0x7FFF)
    rnd = lambda v: (v + half + ((v >> 16) & one)) >> 16
    xe = rnd(bits[:_PACK_ROWS // 2, :])
    xo = rnd(bits[_PACK_ROWS // 2:, :])
    h = _D // 2
    pack_a = xe | (pltpu.roll(xe, h, axis=1) << 16)
    pack_b = pltpu.roll(xo, h, axis=1) | (xo << 16)
    lane = lax.broadcasted_iota(jnp.uint32, pack_a.shape, 1)
    o_ref[...] = lax.bitcast_convert_type(
        jnp.where(lane < h, pack_a, pack_b), jnp.int32)


def _pack_tc(emb):
    packed2 = pl.pallas_call(
        _pack_tc_body,
        out_shape=jax.ShapeDtypeStruct((_NUM_NODES // 2, _D), jnp.int32),
        grid=(_NUM_NODES // _PACK_ROWS,),
        in_specs=[pl.BlockSpec((_PACK_ROWS, _D), lambda i: (i, 0))],
        out_specs=pl.BlockSpec((_PACK_ROWS // 2, _D), lambda i: (i, 0)),
    )(emb)
    return jnp.reshape(packed2, (_NUM_NODES, _D // 2))


def _loss_tc_body(e_ref, z_ref, o_ref):
    e = e_ref[...]
    z = z_ref[...]
    loss_edges = -jnp.mean(jnp.log1p(-jnp.exp(-_EPS - e)))
    loss_non = jnp.mean(z)
    o_ref[0, 0] = (loss_edges + _NEG_SCALE * loss_non) / (1.0 + _NEG_SCALE)


def _loss_tc(edge_dots, non_dots):
    return pl.pallas_call(
        _loss_tc_body,
        out_shape=jax.ShapeDtypeStruct((1, 1), jnp.float32),
        out_specs=pl.BlockSpec(memory_space=pltpu.SMEM),
    )(edge_dots, non_dots)


@jax.jit
def kernel(emb, ones_idx, zeros_idx):
    li = jnp.concatenate([ones_idx[:, 0], zeros_idx[:, 0]]).astype(jnp.int32)
    ri = jnp.concatenate([ones_idx[:, 1], zeros_idx[:, 1]]).astype(jnp.int32)
    half = _PACK_ROWS // 2

    def _pi(n):
        i = n // _PACK_ROWS
        t = n - i * _PACK_ROWS
        return i * _PACK_ROWS + jnp.where(t < half, 2 * t, 2 * t - (_PACK_ROWS - 1))

    dots = _sc_dots(_pack_tc(emb), _pi(li), _pi(ri))
    edge_dots = dots[:_B].reshape(_B // _D, _D)
    non_dots = dots[_B:].reshape(_B // _D, _D)
    return _loss_tc(edge_dots, non_dots)[0, 0]
